# Initial kernel scaffold; baseline (speedup 1.0000x reference)
import jax
import jax.numpy as jnp
from jax.experimental import pallas as pl


def kernel(output_tensor, inputs, W0, W1):
    raise NotImplementedError("write your pallas kernel here")


# speedup vs baseline: 1.0055x; 1.0055x over previous
import functools
import math

import jax
import jax.numpy as jnp
from jax.experimental import pallas as pl

_E = 64
_C = 32
_D = 1024
_F = 4096
_BF = 2048
_NF = _F // _BF


def _ffn_kernel(x_ref, w0_ref, w1_ref, o_ref):
    f = pl.program_id(1)
    x = x_ref[0, 0].astype(jnp.bfloat16)
    w0 = w0_ref[0].astype(jnp.bfloat16)
    h = jax.lax.dot_general(
        x, w0, (((1,), (1,)), ((), ())),
        preferred_element_type=jnp.float32,
    )
    h = 0.5 * h * (1.0 + jax.lax.erf(h * (1.0 / math.sqrt(2.0))))
    h = h.astype(jnp.bfloat16)
    w1 = w1_ref[0].astype(jnp.bfloat16)
    part = jax.lax.dot_general(
        h, w1, (((1,), (1,)), ((), ())),
        preferred_element_type=jnp.float32,
    )

    @pl.when(f == 0)
    def _init():
        o_ref[0, 0] = part

    @pl.when(f != 0)
    def _acc():
        o_ref[0, 0] += part


@functools.partial(jax.jit, static_argnames=())
def _run(inputs, W0, W1):
    g = inputs.shape[0]
    out = pl.pallas_call(
        _ffn_kernel,
        grid=(_E, _NF),
        in_specs=[
            pl.BlockSpec((1, 1, _C, _D), lambda e, f: (0, e, 0, 0)),
            pl.BlockSpec((1, _BF, _D), lambda e, f: (e, f, 0)),
            pl.BlockSpec((1, _D, _BF), lambda e, f: (e, 0, f)),
        ],
        out_specs=pl.BlockSpec((1, 1, _C, _D), lambda e, f: (0, e, 0, 0)),
        out_shape=jax.ShapeDtypeStruct((g, _E, _C, _D), jnp.float32),
    )(inputs, W0, W1)
    return out


def kernel(output_tensor, inputs, W0, W1):
    return _run(inputs, W0, W1)
